# baseline (speedup 1.0000x reference)
import jax
import jax.numpy as jnp
from jax import lax
from jax.experimental import pallas as pl
from jax.experimental.pallas import tpu as pltpu
from jax.experimental.pallas import tpu_sc as plsc

B = 16384
D = 64
NC = 2
NS = 16
L = 16
NW = NC * NS
BPW = B // NW
CHUNK = 128
NCHUNK = BPW // CHUNK


def _fac_body(uid_hbm, mid_hbm, users_hbm, movies_hbm, ub_hbm, mb_hbm,
              out_hbm,
              uidx_v, midx_v, urows_v, mrows_v, ub_v, mb_v, out_v, sem):
    c = lax.axis_index("c")
    s = lax.axis_index("s")
    wid = s * NC + c

    pltpu.sync_copy(uid_hbm.at[wid], uidx_v)
    pltpu.sync_copy(mid_hbm.at[wid], midx_v)

    for j in range(NCHUNK):
        rsl = pl.ds(j * CHUNK, CHUNK)
        pltpu.async_copy(users_hbm.at[uidx_v.at[j]], urows_v.at[rsl], sem).wait()
        pltpu.async_copy(movies_hbm.at[midx_v.at[j]], mrows_v.at[rsl], sem).wait()
        pltpu.async_copy(ub_hbm.at[uidx_v.at[j]], ub_v.at[rsl], sem).wait()
        pltpu.async_copy(mb_hbm.at[midx_v.at[j]], mb_v.at[rsl], sem).wait()

    lane = lax.iota(jnp.int32, L)

    def group(g, carry):
        base = g * L
        outvec = ub_v[pl.ds(base, L)] + mb_v[pl.ds(base, L)]
        for i in range(L):
            r = base + i
            acc = urows_v[r, pl.ds(0, L)] * mrows_v[r, pl.ds(0, L)]
            for k in range(1, D // L):
                acc = acc + urows_v[r, pl.ds(k * L, L)] * mrows_v[r, pl.ds(k * L, L)]
            outvec = jnp.where(lane == i, outvec + jnp.sum(acc), outvec)
        out_v[pl.ds(base, L)] = outvec
        return carry

    lax.fori_loop(0, BPW // L, group, 0)

    pltpu.sync_copy(out_v, out_hbm.at[pl.ds(wid * BPW, BPW)])


def kernel(user_ids, movie_ids, users, movies, user_bias, movie_bias):
    uid = user_ids.astype(jnp.int32).reshape(NW, NCHUNK, CHUNK)
    mid = movie_ids.astype(jnp.int32).reshape(NW, NCHUNK, CHUNK)
    ubf = user_bias.reshape(-1)
    mbf = movie_bias.reshape(-1)

    mesh = plsc.VectorSubcoreMesh(core_axis_name="c", subcore_axis_name="s")
    fn = pl.kernel(
        _fac_body,
        out_type=jax.ShapeDtypeStruct((B,), jnp.float32),
        mesh=mesh,
        compiler_params=pltpu.CompilerParams(
            needs_layout_passes=False, use_tc_tiling_on_sc=False),
        scratch_types=[
            pltpu.VMEM((NCHUNK, CHUNK), jnp.int32),
            pltpu.VMEM((NCHUNK, CHUNK), jnp.int32),
            pltpu.VMEM((BPW, D), jnp.float32),
            pltpu.VMEM((BPW, D), jnp.float32),
            pltpu.VMEM((BPW,), jnp.float32),
            pltpu.VMEM((BPW,), jnp.float32),
            pltpu.VMEM((BPW,), jnp.float32),
            pltpu.SemaphoreType.DMA,
        ],
    )
    return fn(uid, mid, users, movies, ubf, mbf)
